# baseline (speedup 1.0000x reference)
import jax
import jax.numpy as jnp
from jax import lax
from jax.experimental import pallas as pl
from jax.experimental.pallas import tpu as pltpu
from jax.experimental.pallas import tpu_sc as plsc

NUM_NODES = 1000000
EMBED_DIM = 64
BATCH = 16384
LANES = 16

NUM_WORKERS = 32


NB = 32768
GRID = -(-NUM_NODES // NB)
K_TC = 17
S0 = K_TC * NB
S1 = (GRID - 1) * NB
TCOLS = (S1 - S0) // 128
TPW = TCOLS // NUM_WORKERS
MHI_ROWS = (S1 - S0) // 2
PACK_ROWS = GRID * (NB // 2)

_TAIL = GRID - 1


def _pack_body(in_ref, out_ref):
    x = in_ref[...]
    y = jnp.swapaxes(x, 0, 1)
    out_ref[...] = jnp.concatenate([y[: NB // 2], y[NB // 2:]], axis=1)


def _pack(emb_t):
    return pl.pallas_call(
        _pack_body,
        grid=(K_TC + 1,),
        in_specs=[pl.BlockSpec(
            (EMBED_DIM, NB),
            lambda g: (0, jnp.where(g < K_TC, g, _TAIL)))],
        out_specs=pl.BlockSpec(
            (NB // 2, 128),
            lambda g: (jnp.where(g < K_TC, g, _TAIL), 0)),
        out_shape=jax.ShapeDtypeStruct((PACK_ROWS, 128), jnp.float32),
    )(emb_t)


def _scpack_body(emb_hbm, mhi_hbm, tiles, rows, sem_i, sem_o):
    wid = lax.axis_index("s") * 2 + lax.axis_index("c")
    base_tc = wid * TPW

    def col_of(t):
        return pl.multiple_of(S0 + (base_tc + t) * 128, 128)

    def start_in(t, buf):
        pltpu.async_copy(emb_hbm.at[:, pl.ds(col_of(t), 128)],
                         tiles.at[buf], sem_i.at[buf])

    def wait_in(buf):
        pltpu.make_async_copy(emb_hbm.at[:, pl.ds(0, 128)],
                              tiles.at[buf], sem_i.at[buf]).wait()

    def wait_out(buf):
        pltpu.make_async_copy(mhi_hbm.at[pl.ds(0, 64)],
                              rows.at[buf], sem_o.at[buf]).wait()

    start_in(0, 0)
    start_in(1, 1)

    dvecs = [lax.iota(jnp.int32, LANES) + (c % 4) * LANES for c in range(8)]

    def body(t2, carry):
        for par in range(2):
            t = 2 * t2 + par
            wait_in(par)

            @pl.when(t2 > 0)
            def _():
                wait_out(par)

            def inner(k, c2):
                for c in range(8):
                    nvec = jnp.full((LANES,), 2 * k + (1 if c >= 4 else 0),
                                    jnp.int32)
                    v = plsc.load_gather(tiles.at[par], [dvecs[c], nvec])
                    rows[par, k, pl.ds(c * LANES, LANES)] = v
                return c2

            lax.fori_loop(0, 64, inner, 0, unroll=2)
            pltpu.async_copy(rows.at[par],
                             mhi_hbm.at[pl.ds((base_tc + t) * 64, 64)],
                             sem_o.at[par])
            start_in(t + 2, par)
        return carry

    lax.fori_loop(0, TPW // 2, body, 0)
    for par in range(2):
        wait_in(par)
        wait_out(par)


def _scpack(emb_t):
    mesh = plsc.VectorSubcoreMesh(core_axis_name="c", subcore_axis_name="s")
    f = pl.kernel(
        _scpack_body,
        out_type=jax.ShapeDtypeStruct((MHI_ROWS, 128), jnp.float32),
        mesh=mesh,
        compiler_params=pltpu.CompilerParams(needs_layout_passes=False),
        scratch_types=[
            pltpu.VMEM((2, EMBED_DIM, 128), jnp.float32),
            pltpu.VMEM((2, 64, 128), jnp.float32),
            pltpu.SemaphoreType.DMA((2,)),
            pltpu.SemaphoreType.DMA((2,)),
        ],
    )
    return f(emb_t)


PAIRS_PER_WORKER = BATCH // NUM_WORKERS
CHUNK = 128
NUM_CHUNKS = PAIRS_PER_WORKER // CHUNK


def _transform_ids(ids):
    row_lo = (lax.shift_left(lax.shift_right_logical(ids, 15), 14)
              + jnp.bitwise_and(ids, 16383))
    nh = jnp.clip(ids - S0, 0, S1 - S0 - 1)
    row_hi = lax.shift_right_logical(nh, 1)
    in_lo = (ids < S0) | (ids >= S1)
    off = jnp.where(
        in_lo,
        lax.shift_left(jnp.bitwise_and(lax.shift_right_logical(ids, 14), 1), 6),
        lax.shift_left(jnp.bitwise_and(nh, 1), 6))
    return row_lo, row_hi, off, jnp.where(in_lo, 1, 0)


def _sc_body(mlo_hbm, mhi_hbm, src_hbm, dst_hbm, out_hbm,
             ilo_s, ilo_d, ihi_s, ihi_d, off_s, off_d, sel_s, sel_d,
             rls, rld, rhs, rhd, out_v, sem_s, sem_d, sem_hs, sem_hd):
    wid = lax.axis_index("s") * 2 + lax.axis_index("c")
    base0 = wid * PAIRS_PER_WORKER

    lane = lax.iota(jnp.int32, LANES)
    perms = [lane ^ sh for sh in (8, 4, 2, 1)]

    for k in range(NUM_CHUNKS):
        base = base0 + k * CHUNK
        pltpu.sync_copy(src_hbm.at[pl.ds(base, CHUNK)], ilo_s)
        pltpu.sync_copy(dst_hbm.at[pl.ds(base, CHUNK)], ilo_d)
        for i in range(CHUNK // LANES):
            sl = pl.ds(i * LANES, LANES)
            rl, rh, off, sel = _transform_ids(ilo_s[sl])
            ilo_s[sl] = rl
            ihi_s[sl] = rh
            off_s[sl] = off
            sel_s[sl] = sel
            rl, rh, off, sel = _transform_ids(ilo_d[sl])
            ilo_d[sl] = rl
            ihi_d[sl] = rh
            off_d[sl] = off
            sel_d[sl] = sel
        cps = [
            pltpu.async_copy(mlo_hbm.at[ilo_s], rls, sem_s),
            pltpu.async_copy(mlo_hbm.at[ilo_d], rld, sem_d),
            pltpu.async_copy(mhi_hbm.at[ihi_s], rhs, sem_hs),
            pltpu.async_copy(mhi_hbm.at[ihi_d], rhd, sem_hd),
        ]
        for cp in cps:
            cp.wait()

        def block(g, carry):
            so = off_s[pl.ds(g * LANES, LANES)]
            do = off_d[pl.ds(g * LANES, LANES)]
            ss = sel_s[pl.ds(g * LANES, LANES)]
            sd = sel_d[pl.ds(g * LANES, LANES)]
            res = jnp.zeros((LANES,), jnp.float32)
            for w in range(LANES):
                i = g * LANES + w
                ms = lax.broadcast_in_dim(ss[w], (LANES,), ()) > 0
                md = lax.broadcast_in_dim(sd[w], (LANES,), ()) > 0
                acc = jnp.zeros((LANES,), jnp.float32)
                for c in range(EMBED_DIM // LANES):
                    s = jnp.where(ms, rls[i, pl.ds(so[w] + c * LANES, LANES)],
                                  rhs[i, pl.ds(so[w] + c * LANES, LANES)])
                    d = jnp.where(md, rld[i, pl.ds(do[w] + c * LANES, LANES)],
                                  rhd[i, pl.ds(do[w] + c * LANES, LANES)])
                    acc = acc + s * d
                for p in perms:
                    acc = acc + acc[p]
                res = jnp.where(lane == w, acc, res)
            out_v[pl.ds(g * LANES, LANES)] = res
            return carry

        lax.fori_loop(0, CHUNK // LANES, block, 0)
        pltpu.sync_copy(out_v, out_hbm.at[pl.ds(base, CHUNK)])


def _gather_dot(m_lo, m_hi, src, dst):
    mesh = plsc.VectorSubcoreMesh(core_axis_name="c", subcore_axis_name="s")
    f = pl.kernel(
        _sc_body,
        out_type=jax.ShapeDtypeStruct((BATCH,), jnp.float32),
        mesh=mesh,
        compiler_params=pltpu.CompilerParams(needs_layout_passes=False),
        scratch_types=[
            pltpu.VMEM((CHUNK,), jnp.int32),
            pltpu.VMEM((CHUNK,), jnp.int32),
            pltpu.VMEM((CHUNK,), jnp.int32),
            pltpu.VMEM((CHUNK,), jnp.int32),
            pltpu.VMEM((CHUNK,), jnp.int32),
            pltpu.VMEM((CHUNK,), jnp.int32),
            pltpu.VMEM((CHUNK,), jnp.int32),
            pltpu.VMEM((CHUNK,), jnp.int32),
            pltpu.VMEM((CHUNK, 128), jnp.float32),
            pltpu.VMEM((CHUNK, 128), jnp.float32),
            pltpu.VMEM((CHUNK, 128), jnp.float32),
            pltpu.VMEM((CHUNK, 128), jnp.float32),
            pltpu.VMEM((CHUNK,), jnp.float32),
            pltpu.SemaphoreType.DMA,
            pltpu.SemaphoreType.DMA,
            pltpu.SemaphoreType.DMA,
            pltpu.SemaphoreType.DMA,
        ],
    )
    return f(m_lo, m_hi, src, dst)


@jax.jit
def kernel(node_pairs, embeddings):
    src = node_pairs[:, 0].astype(jnp.int32)
    dst = node_pairs[:, 1].astype(jnp.int32)
    emb_t = embeddings.T
    m_hi = _scpack(emb_t)
    m_lo = _pack(emb_t)
    return _gather_dot(m_lo, m_hi, src, dst)


# speedup vs baseline: 5.5832x; 5.5832x over previous
import functools

import jax
import jax.numpy as jnp
from jax import lax
from jax.experimental import pallas as pl
from jax.experimental.pallas import tpu as pltpu
from jax.experimental.pallas import tpu_sc as plsc

NUM_NODES = 1000000
EMBED_DIM = 64
BATCH = 16384


NB = 32768
GRID = -(-NUM_NODES // NB)
PACK_ROWS = GRID * (NB // 2)


def _pack_body(in_ref, out_ref):
    x = in_ref[...]
    y = jnp.swapaxes(x, 0, 1)
    out_ref[...] = jnp.concatenate([y[: NB // 2], y[NB // 2:]], axis=1)


def _pack(emb_t):
    return pl.pallas_call(
        _pack_body,
        grid=(GRID,),
        in_specs=[pl.BlockSpec((EMBED_DIM, NB), lambda g: (0, g))],
        out_specs=pl.BlockSpec((NB // 2, 128), lambda g: (g, 0)),
        out_shape=jax.ShapeDtypeStruct((PACK_ROWS, 128), jnp.float32),
    )(emb_t)


NUM_WORKERS = 32
PAIRS_PER_WORKER = BATCH // NUM_WORKERS
CHUNK = 128
NUM_CHUNKS = PAIRS_PER_WORKER // CHUNK
LANES = 16


def _sc_body(m_hbm, src_hbm, dst_hbm, out_hbm,
             idx_s, idx_d, off_s, off_d, rows_s, rows_d, out_v, sem_s, sem_d):
    wid = lax.axis_index("s") * 2 + lax.axis_index("c")
    base0 = wid * PAIRS_PER_WORKER

    lane = lax.iota(jnp.int32, LANES)
    perms = [lane ^ sh for sh in (8, 4, 2, 1)]

    for k in range(NUM_CHUNKS):
        base = base0 + k * CHUNK
        pltpu.sync_copy(src_hbm.at[pl.ds(base, CHUNK)], idx_s.at[k])
        pltpu.sync_copy(dst_hbm.at[pl.ds(base, CHUNK)], idx_d.at[k])
        for i in range(CHUNK // LANES):
            sl = pl.ds(i * LANES, LANES)
            s_ids = idx_s[k, sl]
            d_ids = idx_d[k, sl]
            idx_s[k, sl] = (
                lax.shift_left(lax.shift_right_logical(s_ids, 15), 14)
                + jnp.bitwise_and(s_ids, 16383))
            idx_d[k, sl] = (
                lax.shift_left(lax.shift_right_logical(d_ids, 15), 14)
                + jnp.bitwise_and(d_ids, 16383))
            off_s[k, sl] = lax.shift_left(
                jnp.bitwise_and(lax.shift_right_logical(s_ids, 14), 1), 6)
            off_d[k, sl] = lax.shift_left(
                jnp.bitwise_and(lax.shift_right_logical(d_ids, 14), 1), 6)

    def issue(k):
        p = k % 2
        return (pltpu.async_copy(m_hbm.at[idx_s.at[k]], rows_s.at[p],
                                 sem_s.at[p]),
                pltpu.async_copy(m_hbm.at[idx_d.at[k]], rows_d.at[p],
                                 sem_d.at[p]))

    cps = {0: issue(0)}
    for k in range(NUM_CHUNKS):
        if k + 1 < NUM_CHUNKS:
            cps[k + 1] = issue(k + 1)
        for cp in cps.pop(k):
            cp.wait()
        p = k % 2

        def block(g, carry, k=k, p=p):
            so = off_s[k, pl.ds(g * LANES, LANES)]
            do = off_d[k, pl.ds(g * LANES, LANES)]
            res = jnp.zeros((LANES,), jnp.float32)
            for w in range(LANES):
                i = g * LANES + w
                acc = jnp.zeros((LANES,), jnp.float32)
                for c in range(EMBED_DIM // LANES):
                    s = rows_s[p, i, pl.ds(so[w] + c * LANES, LANES)]
                    d = rows_d[p, i, pl.ds(do[w] + c * LANES, LANES)]
                    acc = acc + s * d
                for pp in perms:
                    acc = acc + acc[pp]
                res = jnp.where(lane == w, acc, res)
            out_v[pl.ds(k * CHUNK + g * LANES, LANES)] = res
            return carry

        lax.fori_loop(0, CHUNK // LANES, block, 0)

    pltpu.sync_copy(out_v, out_hbm.at[pl.ds(base0, PAIRS_PER_WORKER)])


def _gather_dot(m, src, dst):
    mesh = plsc.VectorSubcoreMesh(core_axis_name="c", subcore_axis_name="s")
    f = pl.kernel(
        _sc_body,
        out_type=jax.ShapeDtypeStruct((BATCH,), jnp.float32),
        mesh=mesh,
        scratch_types=[
            pltpu.VMEM((NUM_CHUNKS, CHUNK), jnp.int32),
            pltpu.VMEM((NUM_CHUNKS, CHUNK), jnp.int32),
            pltpu.VMEM((NUM_CHUNKS, CHUNK), jnp.int32),
            pltpu.VMEM((NUM_CHUNKS, CHUNK), jnp.int32),
            pltpu.VMEM((2, CHUNK, 128), jnp.float32),
            pltpu.VMEM((2, CHUNK, 128), jnp.float32),
            pltpu.VMEM((PAIRS_PER_WORKER,), jnp.float32),
            pltpu.SemaphoreType.DMA((2,)),
            pltpu.SemaphoreType.DMA((2,)),
        ],
    )
    return f(m, src, dst)


@jax.jit
def kernel(node_pairs, embeddings):
    src = node_pairs[:, 0].astype(jnp.int32)
    dst = node_pairs[:, 1].astype(jnp.int32)
    emb_t = embeddings.T
    m = _pack(emb_t)
    return _gather_dot(m, src, dst)


# speedup vs baseline: 5.5879x; 1.0009x over previous
import functools

import jax
import jax.numpy as jnp
from jax import lax
from jax.experimental import pallas as pl
from jax.experimental.pallas import tpu as pltpu
from jax.experimental.pallas import tpu_sc as plsc

NUM_NODES = 1000000
EMBED_DIM = 64
BATCH = 16384


NB = 32768
GRID = -(-NUM_NODES // NB)
PACK_ROWS = GRID * (NB // 2)


def _pack_body(in_ref, out_ref):
    x = in_ref[...]
    y = jnp.swapaxes(x, 0, 1)
    out_ref[...] = jnp.concatenate([y[: NB // 2], y[NB // 2:]], axis=1)


def _pack(emb_t):
    return pl.pallas_call(
        _pack_body,
        grid=(GRID,),
        in_specs=[pl.BlockSpec((EMBED_DIM, NB), lambda g: (0, g))],
        out_specs=pl.BlockSpec((NB // 2, 128), lambda g: (g, 0)),
        out_shape=jax.ShapeDtypeStruct((PACK_ROWS, 128), jnp.float32),
        compiler_params=pltpu.CompilerParams(
            dimension_semantics=("parallel",)),
    )(emb_t)


NUM_WORKERS = 32
PAIRS_PER_WORKER = BATCH // NUM_WORKERS
CHUNK = 128
NUM_CHUNKS = PAIRS_PER_WORKER // CHUNK
LANES = 16


def _sc_body(m_hbm, src_hbm, dst_hbm, out_hbm,
             idx_s, idx_d, off_s, off_d, rows_s, rows_d, out_v, sem_s, sem_d):
    wid = lax.axis_index("s") * 2 + lax.axis_index("c")
    base0 = wid * PAIRS_PER_WORKER

    lane = lax.iota(jnp.int32, LANES)
    perms = [lane ^ sh for sh in (8, 4, 2, 1)]

    for k in range(NUM_CHUNKS):
        base = base0 + k * CHUNK
        pltpu.sync_copy(src_hbm.at[pl.ds(base, CHUNK)], idx_s.at[k])
        pltpu.sync_copy(dst_hbm.at[pl.ds(base, CHUNK)], idx_d.at[k])
        for i in range(CHUNK // LANES):
            sl = pl.ds(i * LANES, LANES)
            s_ids = idx_s[k, sl]
            d_ids = idx_d[k, sl]
            idx_s[k, sl] = (
                lax.shift_left(lax.shift_right_logical(s_ids, 15), 14)
                + jnp.bitwise_and(s_ids, 16383))
            idx_d[k, sl] = (
                lax.shift_left(lax.shift_right_logical(d_ids, 15), 14)
                + jnp.bitwise_and(d_ids, 16383))
            off_s[k, sl] = lax.shift_left(
                jnp.bitwise_and(lax.shift_right_logical(s_ids, 14), 1), 6)
            off_d[k, sl] = lax.shift_left(
                jnp.bitwise_and(lax.shift_right_logical(d_ids, 14), 1), 6)

    def issue(k):
        p = k % 2
        return (pltpu.async_copy(m_hbm.at[idx_s.at[k]], rows_s.at[p],
                                 sem_s.at[p]),
                pltpu.async_copy(m_hbm.at[idx_d.at[k]], rows_d.at[p],
                                 sem_d.at[p]))

    cps = {0: issue(0)}
    for k in range(NUM_CHUNKS):
        if k + 1 < NUM_CHUNKS:
            cps[k + 1] = issue(k + 1)
        for cp in cps.pop(k):
            cp.wait()
        p = k % 2

        def block(g, carry, k=k, p=p):
            so = off_s[k, pl.ds(g * LANES, LANES)]
            do = off_d[k, pl.ds(g * LANES, LANES)]
            res = jnp.zeros((LANES,), jnp.float32)
            for w in range(LANES):
                i = g * LANES + w
                acc = jnp.zeros((LANES,), jnp.float32)
                for c in range(EMBED_DIM // LANES):
                    s = rows_s[p, i, pl.ds(so[w] + c * LANES, LANES)]
                    d = rows_d[p, i, pl.ds(do[w] + c * LANES, LANES)]
                    acc = acc + s * d
                for pp in perms:
                    acc = acc + acc[pp]
                res = jnp.where(lane == w, acc, res)
            out_v[pl.ds(k * CHUNK + g * LANES, LANES)] = res
            return carry

        lax.fori_loop(0, CHUNK // LANES, block, 0)

    pltpu.sync_copy(out_v, out_hbm.at[pl.ds(base0, PAIRS_PER_WORKER)])


def _gather_dot(m, src, dst):
    mesh = plsc.VectorSubcoreMesh(core_axis_name="c", subcore_axis_name="s")
    f = pl.kernel(
        _sc_body,
        out_type=jax.ShapeDtypeStruct((BATCH,), jnp.float32),
        mesh=mesh,
        scratch_types=[
            pltpu.VMEM((NUM_CHUNKS, CHUNK), jnp.int32),
            pltpu.VMEM((NUM_CHUNKS, CHUNK), jnp.int32),
            pltpu.VMEM((NUM_CHUNKS, CHUNK), jnp.int32),
            pltpu.VMEM((NUM_CHUNKS, CHUNK), jnp.int32),
            pltpu.VMEM((2, CHUNK, 128), jnp.float32),
            pltpu.VMEM((2, CHUNK, 128), jnp.float32),
            pltpu.VMEM((PAIRS_PER_WORKER,), jnp.float32),
            pltpu.SemaphoreType.DMA((2,)),
            pltpu.SemaphoreType.DMA((2,)),
        ],
    )
    return f(m, src, dst)


@jax.jit
def kernel(node_pairs, embeddings):
    src = node_pairs[:, 0].astype(jnp.int32)
    dst = node_pairs[:, 1].astype(jnp.int32)
    emb_t = embeddings.T
    m = _pack(emb_t)
    return _gather_dot(m, src, dst)
